# baseline (speedup 1.0000x reference)
import jax
import jax.numpy as jnp
from jax.experimental import pallas as pl
from jax.experimental.pallas import tpu as pltpu

_B = 1
_S = 2048
_T = _B * _S
_D = 1024
_E = 8
_H = 2048
_HB = 4
_HBS = _H // _HB


def _moe_body(x_ref, gw_ref, w1_ref, b1_ref, w2_ref, b2_ref,
              y_ref, tpe_ref, aux_ref, comb_ref):
    e = pl.program_id(0)
    hb = pl.program_id(1)

    @pl.when(jnp.logical_and(e == 0, hb == 0))
    def _route():
        x = x_ref[...]
        logits = jax.lax.dot_general(
            x, gw_ref[...], (((1,), (1,)), ((), ())),
            preferred_element_type=jnp.float32)
        mx = jnp.max(logits, axis=1, keepdims=True)
        ex = jnp.exp(logits - mx)
        probs = ex / jnp.sum(ex, axis=1, keepdims=True)

        iota = jax.lax.broadcasted_iota(jnp.int32, (_T, _E), 1)
        v0 = jnp.max(probs, axis=1, keepdims=True)
        i0 = jnp.min(jnp.where(probs >= v0, iota, _E), axis=1, keepdims=True)
        m0 = iota == i0
        p2 = jnp.where(m0, -1.0, probs)
        v1 = jnp.max(p2, axis=1, keepdims=True)
        i1 = jnp.min(jnp.where(p2 >= v1, iota, _E), axis=1, keepdims=True)
        m1 = iota == i1

        denom = v0 + v1 + 1e-9
        comb = jnp.where(m0, v0 / denom, 0.0) + jnp.where(m1, v1 / denom, 0.0)
        comb_ref[...] = comb

        counts = jnp.sum((m0 | m1).astype(jnp.float32), axis=0, keepdims=True)
        tpe_ref[...] = counts
        m_mean = jnp.mean(probs, axis=0, keepdims=True)
        aux = _E * jnp.sum((counts / _T) * m_mean)
        aux_ref[...] = jnp.reshape(aux, (1, 1))

    x = x_ref[...].astype(jnp.bfloat16)
    h = jax.lax.dot_general(
        x, w1_ref[0].astype(jnp.bfloat16), (((1,), (0,)), ((), ())),
        preferred_element_type=jnp.float32) + b1_ref[0]
    h = jnp.maximum(h, 0.0).astype(jnp.bfloat16)
    o = jax.lax.dot_general(
        h, w2_ref[0].astype(jnp.bfloat16), (((1,), (0,)), ((), ())),
        preferred_element_type=jnp.float32)

    iota = jax.lax.broadcasted_iota(jnp.int32, (_T, _E), 1)
    w = jnp.sum(jnp.where(iota == e, comb_ref[...], 0.0), axis=1,
                keepdims=True)
    contrib = w * o
    contrib = jnp.where(hb == 0, contrib + w * b2_ref[0], contrib)

    @pl.when(jnp.logical_and(e == 0, hb == 0))
    def _init():
        y_ref[...] = contrib

    @pl.when(jnp.logical_or(e > 0, hb > 0))
    def _acc():
        y_ref[...] = y_ref[...] + contrib


def kernel(x, gate_W, W1, b1, W2, b2):
    xt = x.reshape(_T, _D)
    y, tpe, aux = pl.pallas_call(
        _moe_body,
        grid=(_E, _HB),
        in_specs=[
            pl.BlockSpec((_T, _D), lambda e, h: (0, 0)),
            pl.BlockSpec((_E, _D), lambda e, h: (0, 0)),
            pl.BlockSpec((1, _D, _HBS), lambda e, h: (e, 0, h)),
            pl.BlockSpec((1, 1, _HBS), lambda e, h: (e * _HB + h, 0, 0)),
            pl.BlockSpec((1, _HBS, _D), lambda e, h: (e, h, 0)),
            pl.BlockSpec((1, 1, _D), lambda e, h: (e, 0, 0)),
        ],
        out_specs=[
            pl.BlockSpec((_T, _D), lambda e, h: (0, 0)),
            pl.BlockSpec((1, _E), lambda e, h: (0, 0)),
            pl.BlockSpec((1, 1), lambda e, h: (0, 0)),
        ],
        out_shape=[
            jax.ShapeDtypeStruct((_T, _D), jnp.float32),
            jax.ShapeDtypeStruct((1, _E), jnp.float32),
            jax.ShapeDtypeStruct((1, 1), jnp.float32),
        ],
        scratch_shapes=[pltpu.VMEM((_T, _E), jnp.float32)],
        compiler_params=pltpu.CompilerParams(
            dimension_semantics=("arbitrary", "arbitrary")),
    )(xt, gate_W, W1, b1.reshape(_E * _HB, 1, _HBS), W2,
      b2.reshape(_E, 1, _D))
    return (y.reshape(_B, _S, _D), aux[0, 0], tpe[0])


# speedup vs baseline: 1.1022x; 1.1022x over previous
import jax
import jax.numpy as jnp
from jax.experimental import pallas as pl
from jax.experimental.pallas import tpu as pltpu

_B = 1
_S = 2048
_T = _B * _S
_D = 1024
_E = 8
_H = 2048
_HB = 2
_HBS = _H // _HB


def _moe_body(x_ref, gw_ref, w1_ref, b1_ref, w2_ref, b2_ref,
              y_ref, tpe_ref, aux_ref, comb_ref):
    e = pl.program_id(0)
    hb = pl.program_id(1)

    @pl.when(jnp.logical_and(e == 0, hb == 0))
    def _route():
        x = x_ref[...]
        logits = jax.lax.dot_general(
            x, gw_ref[...], (((1,), (1,)), ((), ())),
            preferred_element_type=jnp.float32)
        mx = jnp.max(logits, axis=1, keepdims=True)
        ex = jnp.exp(logits - mx)
        probs = ex / jnp.sum(ex, axis=1, keepdims=True)

        iota = jax.lax.broadcasted_iota(jnp.int32, (_T, _E), 1)
        v0 = jnp.max(probs, axis=1, keepdims=True)
        i0 = jnp.min(jnp.where(probs >= v0, iota, _E), axis=1, keepdims=True)
        m0 = iota == i0
        p2 = jnp.where(m0, -1.0, probs)
        v1 = jnp.max(p2, axis=1, keepdims=True)
        i1 = jnp.min(jnp.where(p2 >= v1, iota, _E), axis=1, keepdims=True)
        m1 = iota == i1

        denom = v0 + v1 + 1e-9
        comb = jnp.where(m0, v0 / denom, 0.0) + jnp.where(m1, v1 / denom, 0.0)
        comb_ref[...] = comb

        counts = jnp.sum((m0 | m1).astype(jnp.float32), axis=0, keepdims=True)
        tpe_ref[...] = counts
        m_mean = jnp.mean(probs, axis=0, keepdims=True)
        aux = _E * jnp.sum((counts / _T) * m_mean)
        aux_ref[...] = jnp.reshape(aux, (1, 1))

    iota = jax.lax.broadcasted_iota(jnp.int32, (_T, _E), 1)
    w = jnp.sum(jnp.where(iota == e, comb_ref[...], 0.0), axis=1,
                keepdims=True)

    x = x_ref[...].astype(jnp.bfloat16)
    h = jax.lax.dot_general(
        x, w1_ref[0].astype(jnp.bfloat16), (((1,), (0,)), ((), ())),
        preferred_element_type=jnp.float32) + b1_ref[0]
    h = (w * jnp.maximum(h, 0.0)).astype(jnp.bfloat16)
    contrib = jax.lax.dot_general(
        h, w2_ref[0].astype(jnp.bfloat16), (((1,), (0,)), ((), ())),
        preferred_element_type=jnp.float32)
    contrib = jnp.where(hb == 0, contrib + w * b2_ref[0], contrib)

    @pl.when(jnp.logical_and(e == 0, hb == 0))
    def _init():
        y_ref[...] = contrib

    @pl.when(jnp.logical_or(e > 0, hb > 0))
    def _acc():
        y_ref[...] = y_ref[...] + contrib


def kernel(x, gate_W, W1, b1, W2, b2):
    xt = x.reshape(_T, _D)
    y, tpe, aux = pl.pallas_call(
        _moe_body,
        grid=(_E, _HB),
        in_specs=[
            pl.BlockSpec((_T, _D), lambda e, h: (0, 0)),
            pl.BlockSpec((_E, _D), lambda e, h: (0, 0)),
            pl.BlockSpec((1, _D, _HBS), lambda e, h: (e, 0, h)),
            pl.BlockSpec((1, 1, _HBS), lambda e, h: (e * _HB + h, 0, 0)),
            pl.BlockSpec((1, _HBS, _D), lambda e, h: (e, h, 0)),
            pl.BlockSpec((1, 1, _D), lambda e, h: (e, 0, 0)),
        ],
        out_specs=[
            pl.BlockSpec((_T, _D), lambda e, h: (0, 0)),
            pl.BlockSpec((1, _E), lambda e, h: (0, 0)),
            pl.BlockSpec((1, 1), lambda e, h: (0, 0)),
        ],
        out_shape=[
            jax.ShapeDtypeStruct((_T, _D), jnp.float32),
            jax.ShapeDtypeStruct((1, _E), jnp.float32),
            jax.ShapeDtypeStruct((1, 1), jnp.float32),
        ],
        scratch_shapes=[pltpu.VMEM((_T, _E), jnp.float32)],
        compiler_params=pltpu.CompilerParams(
            dimension_semantics=("arbitrary", "arbitrary")),
    )(xt, gate_W, W1, b1.reshape(_E * _HB, 1, _HBS), W2,
      b2.reshape(_E, 1, _D))
    return (y.reshape(_B, _S, _D), aux[0, 0], tpe[0])
